# baseline (speedup 1.0000x reference)
import functools

import jax
import jax.numpy as jnp
from jax import lax
from jax.experimental import pallas as pl
from jax.experimental.pallas import tpu as pltpu
from jax.experimental.pallas import tpu_sc as plsc

EPS = 1e-5

NC = 2
NS = 16
LANES = 16


def _rel_body(rel_ref, wrel_ref, sig_ref, newrel_ref):
    r = rel_ref[...]
    sig_ref[...] = jax.nn.sigmoid(r)
    newrel_ref[...] = lax.dot_general(
        r, wrel_ref[...], (((1,), (1,)), ((), ())),
        preferred_element_type=jnp.float32)


def _fin_body(ent_ref, agg2_ref, cnt2_ref, ws_ref, wn_ref, g_ref, b_ref,
              out_ref):
    agg = agg2_ref[0] + agg2_ref[1]
    cnt = cnt2_ref[0, :, :1] + cnt2_ref[1, :, :1]
    agg = agg / jnp.maximum(cnt, 1.0)
    out = lax.dot_general(ent_ref[...], ws_ref[...],
                          (((1,), (1,)), ((), ())),
                          preferred_element_type=jnp.float32)
    out += lax.dot_general(agg, wn_ref[...], (((1,), (1,)), ((), ())),
                           preferred_element_type=jnp.float32)
    mean = jnp.mean(out, axis=0)
    var = jnp.mean((out - mean) ** 2, axis=0)
    out = (out - mean) * lax.rsqrt(var + EPS) * g_ref[...] + b_ref[...]
    out_ref[...] = jnp.maximum(out, 0.0)


def _make_sc_edge_kernel(N, D, E):
    NW = NC * NS
    EPW = E // NW
    C = 80
    NCHUNK = EPW // C
    assert EPW * NW == E and NCHUNK * C == EPW
    assert C % LANES == 0 and C % 8 == 0 and NCHUNK % 2 == 1 and NCHUNK > 3
    BKC = 5
    NBLK = NCHUNK // BKC
    assert NBLK * BKC == NCHUNK
    RPS0 = (N // NS) & ~7
    RPS1 = N - RPS0 * (NS - 1)
    assert RPS0 % 8 == 0 and RPS1 % 8 == 0
    DS = D // LANES

    mesh = plsc.VectorSubcoreMesh(core_axis_name="c", subcore_axis_name="s",
                                  num_cores=NC, num_subcores=NS)

    @functools.partial(
        pl.kernel,
        out_type=[
            jax.ShapeDtypeStruct((NC, N, D), jnp.float32),
            jax.ShapeDtypeStruct((NC, N, D), jnp.float32),
        ],
        mesh=mesh,
        scratch_types=[
            pltpu.VMEM_SHARED((N, D), jnp.float32),
            pltpu.VMEM((2 * C,), jnp.int32),
            pltpu.VMEM((2 * C,), jnp.int32),
            pltpu.VMEM((C,), jnp.int32),
            pltpu.VMEM((C,), jnp.int32),
            pltpu.VMEM((BKC * C,), jnp.int32),
            pltpu.VMEM((C, D), jnp.float32),
            pltpu.VMEM((C, D), jnp.float32),
            pltpu.VMEM((C, D), jnp.float32),
            pltpu.VMEM((C, D), jnp.float32),
            pltpu.SemaphoreType.DMA,
            pltpu.SemaphoreType.DMA,
            pltpu.SemaphoreType.DMA,
            pltpu.SemaphoreType.DMA,
            pltpu.SemaphoreType.DMA,
            pltpu.SemaphoreType.DMA,
            pltpu.SemaphoreType.DMA,
            pltpu.SemaphoreType.DMA,
            pltpu.SemaphoreType.DMA,
        ],
    )
    def sc_edges(ent_hbm, sig_hbm, gidx_hbm, dst_hbm,
                 agg_hbm, cnt_hbm,
                 agg_sp, gidx_a, gidx_b, didx_a, didx_b, dstblk,
                 rows_a, rows_b, sig_a, sig_b,
                 sga, sgb, sda, sdb, g1a, g2a, g1b, g2b, ssc):
        c = lax.axis_index("c")
        s = lax.axis_index("s")
        wid = s * NC + c

        zeros = jnp.zeros((LANES,), jnp.float32)
        ones = jnp.ones((LANES,), jnp.float32)

        def fill_zrow(r, _):
            for j in range(DS):
                rows_a[r, pl.ds(j * LANES, LANES)] = zeros
            return 0
        lax.fori_loop(0, C, fill_zrow, 0)

        row0 = pl.multiple_of(s * RPS0, 8)

        def _zero_stripe(nrows):
            for k in range(nrows // C):
                off = pl.multiple_of(row0 + k * C, 8)
                pltpu.sync_copy(rows_a, agg_sp.at[pl.ds(off, C)])
            if nrows % C:
                off = pl.multiple_of(row0 + nrows - C, 8)
                pltpu.sync_copy(rows_a, agg_sp.at[pl.ds(off, C)])

        @pl.when(s == NS - 1)
        def _zero_last():
            _zero_stripe(RPS1)

        @pl.when(s != NS - 1)
        def _zero_main():
            _zero_stripe(RPS0)

        plsc.subcore_barrier()

        t0 = wid * NCHUNK
        e0 = wid * EPW
        gsets = ((gidx_a, sga), (gidx_b, sgb))
        dsets = ((didx_a, sda), (didx_b, sdb))
        rsets = ((rows_a, sig_a, g1a, g2a), (rows_b, sig_b, g1b, g2b))

        def gidx_issue(i, gs):
            off = pl.multiple_of((t0 + i) * (2 * C), 8)
            pltpu.async_copy(gidx_hbm.at[pl.ds(off, 2 * C)], gs[0], gs[1])

        def gidx_wait(i, gs):
            pltpu.make_async_copy(gidx_hbm.at[pl.ds(0, 2 * C)], gs[0],
                                  gs[1]).wait()

        def didx_issue(i, ds_):
            off = pl.multiple_of(e0 + i * C, 8)
            pltpu.async_copy(dst_hbm.at[pl.ds(off, C)], ds_[0], ds_[1])

        def didx_wait(ds_):
            pltpu.make_async_copy(dst_hbm.at[pl.ds(0, C)], ds_[0],
                                  ds_[1]).wait()

        def gathers_issue(gs, rs):
            gi, _ = gs
            rv, sv, s1, s2 = rs
            pltpu.async_copy(ent_hbm.at[pl.ds(0, C)], rv, s1)
            pltpu.async_copy(ent_hbm.at[pl.ds(C, C)], sv, s2)

        def gathers_wait(gs, rs):
            gi, _ = gs
            rv, sv, s1, s2 = rs
            pltpu.make_async_copy(ent_hbm.at[gi.at[pl.ds(0, C)]], rv,
                                  s1).wait()
            pltpu.make_async_copy(sig_hbm.at[gi.at[pl.ds(C, C)]], sv,
                                  s2).wait()

        def scatter_wait(ds_, rs):
            pltpu.make_async_copy(rs[0], agg_sp.at[ds_[0]], ssc).wait()

        def do_chunk(i, p, first, last):
            q = 1 - p
            gathers_wait(gsets[p], rsets[p])
            if not first:
                scatter_wait(dsets[q], rsets[q])
            if not last:
                @pl.when(i + 2 < NCHUNK)
                def _pf():
                    gidx_issue(i + 2, gsets[p])
                didx_issue(i + 1, dsets[q])
                gidx_wait(i + 1, gsets[q])
                gathers_issue(gsets[q], rsets[q])
            rv, sv = rsets[p][0], rsets[p][1]

            def edge_body(e, _):
                for j in range(DS):
                    sl = pl.ds(j * LANES, LANES)
                    rv[e, sl] = rv[e, sl] * sv[e, sl]
                return 0
            lax.fori_loop(0, C, edge_body, 0)
            didx_wait(dsets[p])
            pltpu.async_copy(rv, agg_sp.at[dsets[p][0]], ssc, add=True)

        gidx_issue(0, gsets[0])
        gidx_issue(1, gsets[1])
        didx_issue(0, dsets[0])
        gidx_wait(0, gsets[0])
        gathers_issue(gsets[0], rsets[0])

        def pair_body(jj, _):
            i = jj * 2

            @pl.when(jj == 0)
            def _p0():
                do_chunk(i, 0, True, False)

            @pl.when(jj != 0)
            def _pn():
                do_chunk(i, 0, False, False)

            do_chunk(i + 1, 1, False, False)
            return 0
        lax.fori_loop(0, NCHUNK // 2, pair_body, 0)
        do_chunk(NCHUNK - 1, 0, False, True)
        scatter_wait(dsets[0], rsets[0])

        plsc.subcore_barrier()

        @pl.when(s == NS - 1)
        def _flush_last():
            pltpu.sync_copy(agg_sp.at[pl.ds(row0, RPS1)],
                            agg_hbm.at[c, pl.ds(row0, RPS1)])

        @pl.when(s != NS - 1)
        def _flush_main():
            pltpu.sync_copy(agg_sp.at[pl.ds(row0, RPS0)],
                            agg_hbm.at[c, pl.ds(row0, RPS0)])

        def fill_zrow2(r, _):
            for j in range(DS):
                rows_a[r, pl.ds(j * LANES, LANES)] = zeros
                sig_a[r, pl.ds(j * LANES, LANES)] = ones
            return 0
        lax.fori_loop(0, C, fill_zrow2, 0)

        @pl.when(s == NS - 1)
        def _zero2_last():
            _zero_stripe(RPS1)

        @pl.when(s != NS - 1)
        def _zero2_main():
            _zero_stripe(RPS0)

        plsc.subcore_barrier()

        def blk_body(bi, _):
            off = pl.multiple_of(e0 + bi * (BKC * C), 8)
            pltpu.sync_copy(dst_hbm.at[pl.ds(off, BKC * C)], dstblk)
            for k in range(BKC):
                pltpu.async_copy(
                    sig_a, agg_sp.at[dstblk.at[pl.ds(k * C, C)]], ssc,
                    add=True)
            for k in range(BKC):
                pltpu.make_async_copy(
                    sig_a, agg_sp.at[dstblk.at[pl.ds(k * C, C)]],
                    ssc).wait()
            return 0
        lax.fori_loop(0, NBLK, blk_body, 0)

        plsc.subcore_barrier()

        @pl.when(s == NS - 1)
        def _flush2_last():
            pltpu.sync_copy(agg_sp.at[pl.ds(row0, RPS1)],
                            cnt_hbm.at[c, pl.ds(row0, RPS1)])

        @pl.when(s != NS - 1)
        def _flush2_main():
            pltpu.sync_copy(agg_sp.at[pl.ds(row0, RPS0)],
                            cnt_hbm.at[c, pl.ds(row0, RPS0)])

    return sc_edges


def kernel(ent_emb, rel_emb, W_self, W_neighbor, W_rel, gamma, beta,
           edge_index, edge_type):
    N, D = ent_emb.shape
    R = rel_emb.shape[0]
    E = edge_type.shape[0]
    src = edge_index[0]
    dst = edge_index[1]

    NW = NC * NS
    EPW = E // NW
    C = 80
    NCHUNK = EPW // C
    gidx_pack = jnp.stack(
        [src.reshape(NW, NCHUNK, C),
         edge_type.reshape(NW, NCHUNK, C)], axis=2).reshape(-1)

    sig_tab, new_rel = pl.pallas_call(
        _rel_body,
        out_shape=[
            jax.ShapeDtypeStruct((R, D), jnp.float32),
            jax.ShapeDtypeStruct((R, D), jnp.float32),
        ],
    )(rel_emb, W_rel)

    sc_edges = _make_sc_edge_kernel(N, D, E)
    agg2, cnt2 = sc_edges(ent_emb, sig_tab, gidx_pack, dst)

    out = pl.pallas_call(
        _fin_body,
        out_shape=jax.ShapeDtypeStruct((N, D), jnp.float32),
    )(ent_emb, agg2, cnt2, W_self, W_neighbor, gamma, beta)

    return (out, new_rel)


# speedup vs baseline: 1.7712x; 1.7712x over previous
import functools

import jax
import jax.numpy as jnp
from jax import lax
from jax.experimental import pallas as pl
from jax.experimental.pallas import tpu as pltpu
from jax.experimental.pallas import tpu_sc as plsc

EPS = 1e-5

NC = 2
NS = 16
LANES = 16


def _rel_body(rel_ref, wrel_ref, sig_ref, newrel_ref):
    r = rel_ref[...]
    sig_ref[...] = jax.nn.sigmoid(r)
    newrel_ref[...] = lax.dot_general(
        r, wrel_ref[...], (((1,), (1,)), ((), ())),
        preferred_element_type=jnp.float32)


def _fin_body(ent_ref, agg2_ref, cnt2_ref, ws_ref, wn_ref, g_ref, b_ref,
              out_ref):
    agg = agg2_ref[0] + agg2_ref[1]
    cnt = cnt2_ref[0, :, :1] + cnt2_ref[1, :, :1]
    agg = agg / jnp.maximum(cnt, 1.0)
    out = lax.dot_general(ent_ref[...], ws_ref[...],
                          (((1,), (1,)), ((), ())),
                          preferred_element_type=jnp.float32)
    out += lax.dot_general(agg, wn_ref[...], (((1,), (1,)), ((), ())),
                           preferred_element_type=jnp.float32)
    mean = jnp.mean(out, axis=0)
    var = jnp.mean((out - mean) ** 2, axis=0)
    out = (out - mean) * lax.rsqrt(var + EPS) * g_ref[...] + b_ref[...]
    out_ref[...] = jnp.maximum(out, 0.0)


def _make_sc_edge_kernel(N, D, E, R_TAB):
    NW = NC * NS
    EPW = E // NW
    C = 80
    NCHUNK = EPW // C
    assert EPW * NW == E and NCHUNK * C == EPW
    assert C % LANES == 0 and C % 8 == 0 and NCHUNK % 2 == 1 and NCHUNK > 3
    BKC = 5
    NBLK = NCHUNK // BKC
    assert NBLK * BKC == NCHUNK
    RPS0 = (N // NS) & ~7
    RPS1 = N - RPS0 * (NS - 1)
    assert RPS0 % 8 == 0 and RPS1 % 8 == 0
    DS = D // LANES

    mesh = plsc.VectorSubcoreMesh(core_axis_name="c", subcore_axis_name="s",
                                  num_cores=NC, num_subcores=NS)

    @functools.partial(
        pl.kernel,
        out_type=[
            jax.ShapeDtypeStruct((NC, N, D), jnp.float32),
            jax.ShapeDtypeStruct((NC, N, D), jnp.float32),
        ],
        mesh=mesh,
        scratch_types=[
            pltpu.VMEM_SHARED((N, D), jnp.float32),
            pltpu.VMEM_SHARED((104, D), jnp.float32),
            pltpu.VMEM((2 * C,), jnp.int32),
            pltpu.VMEM((2 * C,), jnp.int32),
            pltpu.VMEM((C,), jnp.int32),
            pltpu.VMEM((C,), jnp.int32),
            pltpu.VMEM((BKC * C,), jnp.int32),
            pltpu.VMEM((C, D), jnp.float32),
            pltpu.VMEM((C, D), jnp.float32),
            pltpu.VMEM((C, D), jnp.float32),
            pltpu.VMEM((C, D), jnp.float32),
            pltpu.SemaphoreType.DMA,
            pltpu.SemaphoreType.DMA,
            pltpu.SemaphoreType.DMA,
            pltpu.SemaphoreType.DMA,
            pltpu.SemaphoreType.DMA,
            pltpu.SemaphoreType.DMA,
            pltpu.SemaphoreType.DMA,
            pltpu.SemaphoreType.DMA,
            pltpu.SemaphoreType.DMA,
        ],
    )
    def sc_edges(ent_hbm, sig_hbm, gidx_hbm, dst_hbm,
                 agg_hbm, cnt_hbm,
                 agg_sp, sig_sp, gidx_a, gidx_b, didx_a, didx_b, dstblk,
                 rows_a, rows_b, sig_a, sig_b,
                 sga, sgb, sda, sdb, g1a, g2a, g1b, g2b, ssc):
        c = lax.axis_index("c")
        s = lax.axis_index("s")
        wid = s * NC + c

        zeros = jnp.zeros((LANES,), jnp.float32)
        ones = jnp.ones((LANES,), jnp.float32)

        def fill_zrow(r, _):
            for j in range(DS):
                rows_a[r, pl.ds(j * LANES, LANES)] = zeros
            return 0
        lax.fori_loop(0, C, fill_zrow, 0)

        row0 = pl.multiple_of(s * RPS0, 8)

        def _zero_stripe(nrows):
            for k in range(nrows // C):
                off = pl.multiple_of(row0 + k * C, 8)
                pltpu.sync_copy(rows_a, agg_sp.at[pl.ds(off, C)])
            if nrows % C:
                off = pl.multiple_of(row0 + nrows - C, 8)
                pltpu.sync_copy(rows_a, agg_sp.at[pl.ds(off, C)])

        @pl.when(s == NS - 1)
        def _zero_last():
            _zero_stripe(RPS1)

        @pl.when(s != NS - 1)
        def _zero_main():
            _zero_stripe(RPS0)

        @pl.when(s == 0)
        def _stage_sig():
            pltpu.sync_copy(sig_hbm, sig_sp.at[pl.ds(0, R_TAB)])

        plsc.subcore_barrier()

        t0 = wid * NCHUNK
        e0 = wid * EPW
        gsets = ((gidx_a, sga), (gidx_b, sgb))
        dsets = ((didx_a, sda), (didx_b, sdb))
        rsets = ((rows_a, sig_a, g1a, g2a), (rows_b, sig_b, g1b, g2b))

        def gidx_issue(i, gs):
            off = pl.multiple_of((t0 + i) * (2 * C), 8)
            pltpu.async_copy(gidx_hbm.at[pl.ds(off, 2 * C)], gs[0], gs[1])

        def gidx_wait(i, gs):
            pltpu.make_async_copy(gidx_hbm.at[pl.ds(0, 2 * C)], gs[0],
                                  gs[1]).wait()

        def didx_issue(i, ds_):
            off = pl.multiple_of(e0 + i * C, 8)
            pltpu.async_copy(dst_hbm.at[pl.ds(off, C)], ds_[0], ds_[1])

        def didx_wait(ds_):
            pltpu.make_async_copy(dst_hbm.at[pl.ds(0, C)], ds_[0],
                                  ds_[1]).wait()

        def gathers_issue(gs, rs):
            gi, _ = gs
            rv, sv, s1, s2 = rs
            pltpu.async_copy(ent_hbm.at[gi.at[pl.ds(0, C)]], rv, s1)
            pltpu.async_copy(sig_sp.at[gi.at[pl.ds(C, C)]], sv, s2)

        def gathers_wait(gs, rs):
            gi, _ = gs
            rv, sv, s1, s2 = rs
            pltpu.make_async_copy(ent_hbm.at[gi.at[pl.ds(0, C)]], rv,
                                  s1).wait()
            pltpu.make_async_copy(sig_sp.at[gi.at[pl.ds(C, C)]], sv,
                                  s2).wait()

        def scatter_wait(ds_, rs):
            pltpu.make_async_copy(rs[0], agg_sp.at[ds_[0]], ssc).wait()

        def do_chunk(i, p, first, last):
            q = 1 - p
            gathers_wait(gsets[p], rsets[p])
            if not first:
                scatter_wait(dsets[q], rsets[q])
            if not last:
                @pl.when(i + 2 < NCHUNK)
                def _pf():
                    gidx_issue(i + 2, gsets[p])
                didx_issue(i + 1, dsets[q])
                gidx_wait(i + 1, gsets[q])
                gathers_issue(gsets[q], rsets[q])
            rv, sv = rsets[p][0], rsets[p][1]

            def edge_body(e, _):
                for j in range(DS):
                    sl = pl.ds(j * LANES, LANES)
                    rv[e, sl] = rv[e, sl] * sv[e, sl]
                return 0
            lax.fori_loop(0, C, edge_body, 0)
            didx_wait(dsets[p])
            pltpu.async_copy(rv, agg_sp.at[dsets[p][0]], ssc, add=True)

        gidx_issue(0, gsets[0])
        gidx_issue(1, gsets[1])
        didx_issue(0, dsets[0])
        gidx_wait(0, gsets[0])
        gathers_issue(gsets[0], rsets[0])

        def pair_body(jj, _):
            i = jj * 2

            @pl.when(jj == 0)
            def _p0():
                do_chunk(i, 0, True, False)

            @pl.when(jj != 0)
            def _pn():
                do_chunk(i, 0, False, False)

            do_chunk(i + 1, 1, False, False)
            return 0
        lax.fori_loop(0, NCHUNK // 2, pair_body, 0)
        do_chunk(NCHUNK - 1, 0, False, True)
        scatter_wait(dsets[0], rsets[0])

        plsc.subcore_barrier()

        @pl.when(s == NS - 1)
        def _flush_last():
            pltpu.sync_copy(agg_sp.at[pl.ds(row0, RPS1)],
                            agg_hbm.at[c, pl.ds(row0, RPS1)])

        @pl.when(s != NS - 1)
        def _flush_main():
            pltpu.sync_copy(agg_sp.at[pl.ds(row0, RPS0)],
                            agg_hbm.at[c, pl.ds(row0, RPS0)])

        def fill_zrow2(r, _):
            for j in range(DS):
                rows_a[r, pl.ds(j * LANES, LANES)] = zeros
                sig_a[r, pl.ds(j * LANES, LANES)] = ones
            return 0
        lax.fori_loop(0, C, fill_zrow2, 0)

        @pl.when(s == NS - 1)
        def _zero2_last():
            _zero_stripe(RPS1)

        @pl.when(s != NS - 1)
        def _zero2_main():
            _zero_stripe(RPS0)

        plsc.subcore_barrier()

        def blk_body(bi, _):
            off = pl.multiple_of(e0 + bi * (BKC * C), 8)
            pltpu.sync_copy(dst_hbm.at[pl.ds(off, BKC * C)], dstblk)
            for k in range(BKC):
                pltpu.async_copy(
                    sig_a, agg_sp.at[dstblk.at[pl.ds(k * C, C)]], ssc,
                    add=True)
            for k in range(BKC):
                pltpu.make_async_copy(
                    sig_a, agg_sp.at[dstblk.at[pl.ds(k * C, C)]],
                    ssc).wait()
            return 0
        lax.fori_loop(0, NBLK, blk_body, 0)

        plsc.subcore_barrier()

        @pl.when(s == NS - 1)
        def _flush2_last():
            pltpu.sync_copy(agg_sp.at[pl.ds(row0, RPS1)],
                            cnt_hbm.at[c, pl.ds(row0, RPS1)])

        @pl.when(s != NS - 1)
        def _flush2_main():
            pltpu.sync_copy(agg_sp.at[pl.ds(row0, RPS0)],
                            cnt_hbm.at[c, pl.ds(row0, RPS0)])

    return sc_edges


def kernel(ent_emb, rel_emb, W_self, W_neighbor, W_rel, gamma, beta,
           edge_index, edge_type):
    N, D = ent_emb.shape
    R = rel_emb.shape[0]
    E = edge_type.shape[0]
    src = edge_index[0]
    dst = edge_index[1]

    NW = NC * NS
    EPW = E // NW
    C = 80
    NCHUNK = EPW // C
    gidx_pack = jnp.stack(
        [src.reshape(NW, NCHUNK, C),
         edge_type.reshape(NW, NCHUNK, C)], axis=2).reshape(-1)

    sig_tab, new_rel = pl.pallas_call(
        _rel_body,
        out_shape=[
            jax.ShapeDtypeStruct((R, D), jnp.float32),
            jax.ShapeDtypeStruct((R, D), jnp.float32),
        ],
    )(rel_emb, W_rel)

    sc_edges = _make_sc_edge_kernel(N, D, E, R)
    agg2, cnt2 = sc_edges(ent_emb, sig_tab, gidx_pack, dst)

    out = pl.pallas_call(
        _fin_body,
        out_shape=jax.ShapeDtypeStruct((N, D), jnp.float32),
    )(ent_emb, agg2, cnt2, W_self, W_neighbor, gamma, beta)

    return (out, new_rel)


# speedup vs baseline: 1.7716x; 1.0002x over previous
import functools

import jax
import jax.numpy as jnp
from jax import lax
from jax.experimental import pallas as pl
from jax.experimental.pallas import tpu as pltpu
from jax.experimental.pallas import tpu_sc as plsc

EPS = 1e-5

NC = 2
NS = 16
LANES = 16


def _rel_body(rel_ref, wrel_ref, sig_ref, newrel_ref):
    r = rel_ref[...]
    sig_ref[...] = jax.nn.sigmoid(r)
    newrel_ref[...] = lax.dot_general(
        r, wrel_ref[...], (((1,), (1,)), ((), ())),
        preferred_element_type=jnp.float32)


def _fin_body(ent_ref, agg2_ref, cnt2_ref, ws_ref, wn_ref, g_ref, b_ref,
              out_ref):
    agg = agg2_ref[0] + agg2_ref[1]
    cnt = cnt2_ref[0, :, :1] + cnt2_ref[1, :, :1]
    agg = agg / jnp.maximum(cnt, 1.0)
    out = lax.dot_general(ent_ref[...], ws_ref[...],
                          (((1,), (1,)), ((), ())),
                          preferred_element_type=jnp.float32)
    out += lax.dot_general(agg, wn_ref[...], (((1,), (1,)), ((), ())),
                           preferred_element_type=jnp.float32)
    mean = jnp.mean(out, axis=0)
    var = jnp.mean((out - mean) ** 2, axis=0)
    out = (out - mean) * lax.rsqrt(var + EPS) * g_ref[...] + b_ref[...]
    out_ref[...] = jnp.maximum(out, 0.0)


def _make_sc_edge_kernel(N, D, E, R_TAB):
    NW = NC * NS
    EPW = E // NW
    C = 80
    NCHUNK = EPW // C
    assert EPW * NW == E and NCHUNK * C == EPW
    assert C % LANES == 0 and C % 8 == 0 and NCHUNK % 2 == 1 and NCHUNK > 3
    BKC = 5
    NBLK = NCHUNK // BKC
    assert NBLK * BKC == NCHUNK
    RPS0 = (N // NS) & ~7
    RPS1 = N - RPS0 * (NS - 1)
    assert RPS0 % 8 == 0 and RPS1 % 8 == 0
    DS = D // LANES

    mesh = plsc.VectorSubcoreMesh(core_axis_name="c", subcore_axis_name="s",
                                  num_cores=NC, num_subcores=NS)

    @functools.partial(
        pl.kernel,
        out_type=[
            jax.ShapeDtypeStruct((NC, N, D), jnp.float32),
            jax.ShapeDtypeStruct((NC, N, D), jnp.float32),
        ],
        mesh=mesh,
        scratch_types=[
            pltpu.VMEM_SHARED((N, D), jnp.float32),
            pltpu.VMEM_SHARED((104, D), jnp.float32),
            pltpu.VMEM((2 * C,), jnp.int32),
            pltpu.VMEM((2 * C,), jnp.int32),
            pltpu.VMEM((C,), jnp.int32),
            pltpu.VMEM((C,), jnp.int32),
            pltpu.VMEM((BKC * C,), jnp.int32),
            pltpu.VMEM((C, D), jnp.float32),
            pltpu.VMEM((C, D), jnp.float32),
            pltpu.VMEM((C, D), jnp.float32),
            pltpu.VMEM((C, D), jnp.float32),
            pltpu.SemaphoreType.DMA,
            pltpu.SemaphoreType.DMA,
            pltpu.SemaphoreType.DMA,
            pltpu.SemaphoreType.DMA,
            pltpu.SemaphoreType.DMA,
            pltpu.SemaphoreType.DMA,
            pltpu.SemaphoreType.DMA,
            pltpu.SemaphoreType.DMA,
            pltpu.SemaphoreType.DMA,
        ],
    )
    def sc_edges(ent_hbm, sig_hbm, gidx_hbm, dst_hbm,
                 agg_hbm, cnt_hbm,
                 agg_sp, sig_sp, gidx_a, gidx_b, didx_a, didx_b, dstblk,
                 rows_a, rows_b, sig_a, sig_b,
                 sga, sgb, sda, sdb, g1a, g2a, g1b, g2b, ssc):
        c = lax.axis_index("c")
        s = lax.axis_index("s")
        wid = s * NC + c

        zeros = jnp.zeros((LANES,), jnp.float32)
        ones = jnp.ones((LANES,), jnp.float32)

        def fill_zrow(r, _):
            for j in range(DS):
                rows_a[r, pl.ds(j * LANES, LANES)] = zeros
            return 0
        lax.fori_loop(0, C, fill_zrow, 0)

        row0 = pl.multiple_of(s * RPS0, 8)

        def _zero_stripe(nrows):
            nfull = nrows // C
            offs = [row0 + k * C for k in range(nfull)]
            if nrows % C:
                offs.append(row0 + nrows - C)
            for off in offs:
                pltpu.async_copy(rows_a, agg_sp.at[pl.ds(
                    pl.multiple_of(off, 8), C)], ssc)
            for off in offs:
                pltpu.make_async_copy(rows_a, agg_sp.at[pl.ds(
                    pl.multiple_of(off, 8), C)], ssc).wait()

        @pl.when(s == NS - 1)
        def _zero_last():
            _zero_stripe(RPS1)

        @pl.when(s != NS - 1)
        def _zero_main():
            _zero_stripe(RPS0)

        @pl.when(s == 0)
        def _stage_sig():
            pltpu.sync_copy(sig_hbm, sig_sp.at[pl.ds(0, R_TAB)])

        plsc.subcore_barrier()

        t0 = wid * NCHUNK
        e0 = wid * EPW
        gsets = ((gidx_a, sga), (gidx_b, sgb))
        dsets = ((didx_a, sda), (didx_b, sdb))
        rsets = ((rows_a, sig_a, g1a, g2a), (rows_b, sig_b, g1b, g2b))

        def gidx_issue(i, gs):
            off = pl.multiple_of((t0 + i) * (2 * C), 8)
            pltpu.async_copy(gidx_hbm.at[pl.ds(off, 2 * C)], gs[0], gs[1])

        def gidx_wait(i, gs):
            pltpu.make_async_copy(gidx_hbm.at[pl.ds(0, 2 * C)], gs[0],
                                  gs[1]).wait()

        def didx_issue(i, ds_):
            off = pl.multiple_of(e0 + i * C, 8)
            pltpu.async_copy(dst_hbm.at[pl.ds(off, C)], ds_[0], ds_[1])

        def didx_wait(ds_):
            pltpu.make_async_copy(dst_hbm.at[pl.ds(0, C)], ds_[0],
                                  ds_[1]).wait()

        def gathers_issue(gs, rs):
            gi, _ = gs
            rv, sv, s1, s2 = rs
            pltpu.async_copy(ent_hbm.at[gi.at[pl.ds(0, C)]], rv, s1)
            pltpu.async_copy(sig_sp.at[gi.at[pl.ds(C, C)]], sv, s2)

        def gathers_wait(gs, rs):
            gi, _ = gs
            rv, sv, s1, s2 = rs
            pltpu.make_async_copy(ent_hbm.at[gi.at[pl.ds(0, C)]], rv,
                                  s1).wait()
            pltpu.make_async_copy(sig_sp.at[gi.at[pl.ds(C, C)]], sv,
                                  s2).wait()

        def scatter_wait(ds_, rs):
            pltpu.make_async_copy(rs[0], agg_sp.at[ds_[0]], ssc).wait()

        def do_chunk(i, p, first, last):
            q = 1 - p
            gathers_wait(gsets[p], rsets[p])
            if not first:
                scatter_wait(dsets[q], rsets[q])
            if not last:
                @pl.when(i + 2 < NCHUNK)
                def _pf():
                    gidx_issue(i + 2, gsets[p])
                didx_issue(i + 1, dsets[q])
                gidx_wait(i + 1, gsets[q])
                gathers_issue(gsets[q], rsets[q])
            rv, sv = rsets[p][0], rsets[p][1]

            def edge_body(e, _):
                for j in range(DS):
                    sl = pl.ds(j * LANES, LANES)
                    rv[e, sl] = rv[e, sl] * sv[e, sl]
                return 0
            lax.fori_loop(0, C, edge_body, 0)
            didx_wait(dsets[p])
            pltpu.async_copy(rv, agg_sp.at[dsets[p][0]], ssc, add=True)

        gidx_issue(0, gsets[0])
        gidx_issue(1, gsets[1])
        didx_issue(0, dsets[0])
        gidx_wait(0, gsets[0])
        gathers_issue(gsets[0], rsets[0])

        def pair_body(jj, _):
            i = jj * 2

            @pl.when(jj == 0)
            def _p0():
                do_chunk(i, 0, True, False)

            @pl.when(jj != 0)
            def _pn():
                do_chunk(i, 0, False, False)

            do_chunk(i + 1, 1, False, False)
            return 0
        lax.fori_loop(0, NCHUNK // 2, pair_body, 0)
        do_chunk(NCHUNK - 1, 0, False, True)
        scatter_wait(dsets[0], rsets[0])

        plsc.subcore_barrier()

        @pl.when(s == NS - 1)
        def _flush_last():
            pltpu.sync_copy(agg_sp.at[pl.ds(row0, RPS1)],
                            agg_hbm.at[c, pl.ds(row0, RPS1)])

        @pl.when(s != NS - 1)
        def _flush_main():
            pltpu.sync_copy(agg_sp.at[pl.ds(row0, RPS0)],
                            agg_hbm.at[c, pl.ds(row0, RPS0)])

        def fill_zrow2(r, _):
            for j in range(DS):
                rows_a[r, pl.ds(j * LANES, LANES)] = zeros
                sig_a[r, pl.ds(j * LANES, LANES)] = ones
            return 0
        lax.fori_loop(0, C, fill_zrow2, 0)

        @pl.when(s == NS - 1)
        def _zero2_last():
            _zero_stripe(RPS1)

        @pl.when(s != NS - 1)
        def _zero2_main():
            _zero_stripe(RPS0)

        plsc.subcore_barrier()

        def blk_body(bi, _):
            off = pl.multiple_of(e0 + bi * (BKC * C), 8)
            pltpu.sync_copy(dst_hbm.at[pl.ds(off, BKC * C)], dstblk)
            for k in range(BKC):
                pltpu.async_copy(
                    sig_a, agg_sp.at[dstblk.at[pl.ds(k * C, C)]], ssc,
                    add=True)
            for k in range(BKC):
                pltpu.make_async_copy(
                    sig_a, agg_sp.at[dstblk.at[pl.ds(k * C, C)]],
                    ssc).wait()
            return 0
        lax.fori_loop(0, NBLK, blk_body, 0)

        plsc.subcore_barrier()

        @pl.when(s == NS - 1)
        def _flush2_last():
            pltpu.sync_copy(agg_sp.at[pl.ds(row0, RPS1)],
                            cnt_hbm.at[c, pl.ds(row0, RPS1)])

        @pl.when(s != NS - 1)
        def _flush2_main():
            pltpu.sync_copy(agg_sp.at[pl.ds(row0, RPS0)],
                            cnt_hbm.at[c, pl.ds(row0, RPS0)])

    return sc_edges


def kernel(ent_emb, rel_emb, W_self, W_neighbor, W_rel, gamma, beta,
           edge_index, edge_type):
    N, D = ent_emb.shape
    R = rel_emb.shape[0]
    E = edge_type.shape[0]
    src = edge_index[0]
    dst = edge_index[1]

    NW = NC * NS
    EPW = E // NW
    C = 80
    NCHUNK = EPW // C
    gidx_pack = jnp.stack(
        [src.reshape(NW, NCHUNK, C),
         edge_type.reshape(NW, NCHUNK, C)], axis=2).reshape(-1)

    sig_tab, new_rel = pl.pallas_call(
        _rel_body,
        out_shape=[
            jax.ShapeDtypeStruct((R, D), jnp.float32),
            jax.ShapeDtypeStruct((R, D), jnp.float32),
        ],
    )(rel_emb, W_rel)

    sc_edges = _make_sc_edge_kernel(N, D, E, R)
    agg2, cnt2 = sc_edges(ent_emb, sig_tab, gidx_pack, dst)

    out = pl.pallas_call(
        _fin_body,
        out_shape=jax.ShapeDtypeStruct((N, D), jnp.float32),
    )(ent_emb, agg2, cnt2, W_self, W_neighbor, gamma, beta)

    return (out, new_rel)
